# baseline (speedup 1.0000x reference)
import functools

import jax
import jax.numpy as jnp
from jax import lax
from jax.experimental import pallas as pl
from jax.experimental.pallas import tpu as pltpu
from jax.experimental.pallas import tpu_sc as plsc

KN = 3
B = 4
C = 128
N = 2048
R = 1024
NBLK = N // R
NB_C = 512


def _topk_body(xb_ref, xl_ref, xt_ref, idx_ref):
    b = pl.program_id(0)
    xb = xb_ref[0]
    xl = xl_ref[0]
    dot = lax.dot_general(
        xl.astype(jnp.bfloat16), xb.astype(jnp.bfloat16),
        (((0,), (0,)), ((), ())),
        preferred_element_type=jnp.float32)
    i = pl.program_id(1)
    nm = jnp.sum(xb * xb, axis=0, keepdims=True)
    nl = jnp.sum(xl * xl, axis=0)[:, None]
    iota_f = lax.broadcasted_iota(jnp.int32, (R, N), 1).astype(jnp.float32)
    row_f = (lax.broadcasted_iota(jnp.int32, (R, 1), 0).astype(jnp.float32)
             + (i * R).astype(jnp.float32))
    dist = jnp.where(iota_f == row_f, jnp.inf,
                     (nl + nm) - 2.0 * dot)
    for k in range(KN - 1):
        mval = jnp.min(dist, axis=1, keepdims=True)
        cand = jnp.where(dist == mval, iota_f, float(N))
        ind_f = jnp.min(cand, axis=1)
        idx_ref[k, 0, 0, :] = ind_f.astype(jnp.int32) + b * N
        if k + 2 < KN:
            dist = jnp.where(cand == ind_f[:, None], jnp.inf, dist)
    xt_ref[...] = xl.T


def _topk_stage(x):
    return pl.pallas_call(
        _topk_body,
        grid=(B, NBLK),
        in_specs=[
            pl.BlockSpec((1, C, N), lambda b, i: (b, 0, 0)),
            pl.BlockSpec((1, C, R), lambda b, i: (b, 0, i)),
        ],
        out_specs=[
            pl.BlockSpec((R, C), lambda b, i: (b * NBLK + i, 0)),
            pl.BlockSpec((KN - 1, 1, 1, R),
                         lambda b, i: (0, b * NBLK + i, 0, 0)),
        ],
        out_shape=[
            jax.ShapeDtypeStruct((B * N, C), jnp.float32),
            jax.ShapeDtypeStruct((KN - 1, B * NBLK, 1, R), jnp.int32),
        ],
    )(x, x)


_NC = 2
_NS = 16
_NW = _NC * _NS
_TOTAL = (KN - 1) * B * N
_PER_W = _TOTAL // _NW
_CH = 128
_NCHUNK = _PER_W // _CH


def _gather_stage(table, idx_flat):
    mesh = plsc.VectorSubcoreMesh(core_axis_name="c", subcore_axis_name="s")

    @functools.partial(
        pl.kernel,
        mesh=mesh,
        out_type=jax.ShapeDtypeStruct((_TOTAL, C), jnp.float32),
        scratch_types=[
            pltpu.VMEM((_PER_W,), jnp.int32),
            pltpu.VMEM((_PER_W, C), jnp.float32),
            pltpu.SemaphoreType.DMA,
            pltpu.SemaphoreType.DMA,
        ],
    )
    def gather_kernel(table_hbm, idx_hbm, out_hbm, idx_v, rows_v, sem, sem2):
        wid = lax.axis_index("s") * _NC + lax.axis_index("c")
        base = wid * _PER_W
        pltpu.sync_copy(idx_hbm.at[pl.ds(base, _PER_W)], idx_v)
        copies = [
            pltpu.async_copy(
                table_hbm.at[idx_v.at[pl.ds(ci * _CH, _CH)]],
                rows_v.at[pl.ds(ci * _CH, _CH)], sem)
            for ci in range(_NCHUNK)
        ]
        outs = []
        for ci, cp in enumerate(copies):
            cp.wait()
            outs.append(pltpu.async_copy(
                rows_v.at[pl.ds(ci * _CH, _CH)],
                out_hbm.at[pl.ds(base + ci * _CH, _CH)], sem2))
        for oc in outs:
            oc.wait()

    return gather_kernel(table, idx_flat)


def _conv_body(t_ref, g_ref, w_ref, b_ref, out_ref):
    acc = lax.dot_general(
        w_ref[0], t_ref[0], (((1,), (1,)), ((), ())),
        preferred_element_type=jnp.float32,
        precision=lax.Precision.HIGHEST)
    for k in range(1, KN):
        acc = acc + lax.dot_general(
            w_ref[k], g_ref[k - 1, 0], (((1,), (1,)), ((), ())),
            preferred_element_type=jnp.float32,
            precision=lax.Precision.HIGHEST)
    out_ref[0] = jnp.maximum(acc + b_ref[0][:, None], 0.0)


def _conv_stage(t3, g, wt, bias):
    return pl.pallas_call(
        _conv_body,
        grid=(B, N // NB_C),
        in_specs=[
            pl.BlockSpec((1, NB_C, C), lambda b, j: (b, j, 0)),
            pl.BlockSpec((KN - 1, 1, NB_C, C), lambda b, j: (0, b, j, 0)),
            pl.BlockSpec((KN, C, C), lambda b, j: (0, 0, 0)),
            pl.BlockSpec((1, C), lambda b, j: (0, 0)),
        ],
        out_specs=pl.BlockSpec((1, C, NB_C), lambda b, j: (b, 0, j)),
        out_shape=jax.ShapeDtypeStruct((B, C, N), jnp.float32),
    )(t3, g, wt, bias)


def kernel(x, W, b):
    table, idx = _topk_stage(x)
    idx_flat = idx.reshape(_TOTAL)
    g = _gather_stage(table, idx_flat)
    g4 = g.reshape(KN - 1, B, N, C)
    t3 = table.reshape(B, N, C)
    wt = jnp.transpose(W, (2, 0, 1))
    bias2 = b.reshape(1, C)
    return _conv_stage(t3, g4, wt, bias2)


# speedup vs baseline: 1.1094x; 1.1094x over previous
import functools

import jax
import jax.numpy as jnp
from jax import lax
from jax.experimental import pallas as pl
from jax.experimental.pallas import tpu as pltpu
from jax.experimental.pallas import tpu_sc as plsc

KN = 3
B = 4
C = 128
N = 2048
R = 512
NBLK = N // R
NB_C = 512


def _topk_body(xb_ref, xl_ref, xt_ref, idx_ref):
    b = pl.program_id(0)
    xb = xb_ref[0]
    xl = xl_ref[0]
    dot2 = lax.dot_general(
        (2.0 * xl).astype(jnp.bfloat16), xb.astype(jnp.bfloat16),
        (((0,), (0,)), ((), ())),
        preferred_element_type=jnp.float32)
    i = pl.program_id(1)
    nm = jnp.sum(xb * xb, axis=0, keepdims=True)
    nl = jnp.sum(xl * xl, axis=0)[:, None]
    iota_f = lax.broadcasted_iota(jnp.int32, (R, N), 1).astype(jnp.float32)
    row_f = (lax.broadcasted_iota(jnp.int32, (R, 1), 0).astype(jnp.float32)
             + (i * R).astype(jnp.float32))
    dist = jnp.where(iota_f == row_f, jnp.inf,
                     (nl + nm) - dot2)
    inds = []
    for k in range(KN - 1):
        mval = jnp.min(dist, axis=1, keepdims=True)
        cand = jnp.where(dist == mval, iota_f, float(N))
        ind_f = jnp.min(cand, axis=1, keepdims=True)
        inds.append(ind_f)
        if k + 2 < KN:
            dist = jnp.where(cand == ind_f, jnp.inf, dist)
    ind_all = jnp.concatenate(inds, axis=1).T
    idx_ref[:, 0, 0, :] = ind_all.astype(jnp.int32) + b * N
    xt_ref[...] = xl.T


def _topk_stage(x):
    return pl.pallas_call(
        _topk_body,
        grid=(B, NBLK),
        in_specs=[
            pl.BlockSpec((1, C, N), lambda b, i: (b, 0, 0)),
            pl.BlockSpec((1, C, R), lambda b, i: (b, 0, i)),
        ],
        out_specs=[
            pl.BlockSpec((R, C), lambda b, i: (b * NBLK + i, 0)),
            pl.BlockSpec((KN - 1, 1, 1, R),
                         lambda b, i: (0, b * NBLK + i, 0, 0)),
        ],
        out_shape=[
            jax.ShapeDtypeStruct((B * N, C), jnp.float32),
            jax.ShapeDtypeStruct((KN - 1, B * NBLK, 1, R), jnp.int32),
        ],
    )(x, x)


_NC = 2
_NS = 16
_NW = _NC * _NS
_TOTAL = (KN - 1) * B * N
_PER_W = _TOTAL // _NW
_CH = 128
_NCHUNK = _PER_W // _CH


def _gather_stage(table, idx_flat):
    mesh = plsc.VectorSubcoreMesh(core_axis_name="c", subcore_axis_name="s")

    @functools.partial(
        pl.kernel,
        mesh=mesh,
        out_type=jax.ShapeDtypeStruct((_TOTAL, C), jnp.float32),
        scratch_types=[
            pltpu.VMEM((_PER_W,), jnp.int32),
            pltpu.VMEM((_PER_W, C), jnp.float32),
            pltpu.SemaphoreType.DMA,
            pltpu.SemaphoreType.DMA,
        ],
    )
    def gather_kernel(table_hbm, idx_hbm, out_hbm, idx_v, rows_v, sem, sem2):
        wid = lax.axis_index("s") * _NC + lax.axis_index("c")
        base = wid * _PER_W
        pltpu.sync_copy(idx_hbm.at[pl.ds(base, _PER_W)], idx_v)
        copies = [
            pltpu.async_copy(
                table_hbm.at[idx_v.at[pl.ds(ci * _CH, _CH)]],
                rows_v.at[pl.ds(ci * _CH, _CH)], sem)
            for ci in range(_NCHUNK)
        ]
        outs = []
        for ci, cp in enumerate(copies):
            cp.wait()
            outs.append(pltpu.async_copy(
                rows_v.at[pl.ds(ci * _CH, _CH)],
                out_hbm.at[pl.ds(base + ci * _CH, _CH)], sem2))
        for oc in outs:
            oc.wait()

    return gather_kernel(table, idx_flat)


def _conv_body(t_ref, g_ref, w_ref, b_ref, out_ref):
    acc = lax.dot_general(
        w_ref[0], t_ref[0], (((1,), (1,)), ((), ())),
        preferred_element_type=jnp.float32,
        precision=lax.Precision.HIGHEST)
    for k in range(1, KN):
        acc = acc + lax.dot_general(
            w_ref[k], g_ref[k - 1, 0], (((1,), (1,)), ((), ())),
            preferred_element_type=jnp.float32,
            precision=lax.Precision.HIGHEST)
    out_ref[0] = jnp.maximum(acc + b_ref[0][:, None], 0.0)


def _conv_stage(t3, g, wt, bias):
    return pl.pallas_call(
        _conv_body,
        grid=(B, N // NB_C),
        in_specs=[
            pl.BlockSpec((1, NB_C, C), lambda b, j: (b, j, 0)),
            pl.BlockSpec((KN - 1, 1, NB_C, C), lambda b, j: (0, b, j, 0)),
            pl.BlockSpec((KN, C, C), lambda b, j: (0, 0, 0)),
            pl.BlockSpec((1, C), lambda b, j: (0, 0)),
        ],
        out_specs=pl.BlockSpec((1, C, NB_C), lambda b, j: (b, 0, j)),
        out_shape=jax.ShapeDtypeStruct((B, C, N), jnp.float32),
    )(t3, g, wt, bias)


def kernel(x, W, b):
    table, idx = _topk_stage(x)
    idx_flat = idx.reshape(_TOTAL)
    g = _gather_stage(table, idx_flat)
    g4 = g.reshape(KN - 1, B, N, C)
    t3 = table.reshape(B, N, C)
    wt = jnp.transpose(W, (2, 0, 1))
    bias2 = b.reshape(1, C)
    return _conv_stage(t3, g4, wt, bias2)
